# Initial kernel scaffold; baseline (speedup 1.0000x reference)
import jax
import jax.numpy as jnp
from jax.experimental import pallas as pl


def kernel(features, edge_index, W1, b1, W2, b2):
    raise NotImplementedError("write your pallas kernel here")


# speedup vs baseline: 6.0628x; 6.0628x over previous
import functools

import jax
import jax.numpy as jnp
from jax import lax
from jax.experimental import pallas as pl
from jax.experimental.pallas import tpu as pltpu
from jax.experimental.pallas import tpu_sc as plsc

N = 10000
E = 320000
D = 128
NC = 2
NS = 16
NW = NC * NS
EPW = E // NW
CH = 80
NCH = EPW // CH
IC = 80
NIC = N // IC
ICL = -(-NIC // NS)

_MESH = plsc.VectorSubcoreMesh(
    core_axis_name="c", subcore_axis_name="s", num_cores=NC, num_subcores=NS)


NP = 10240
RPSD = NP // NS
EPC = E // NS
NCHD = EPC // CH
NRM = NP // NC // NS


def _rsqrt16(d):
    i = lax.bitcast_convert_type(d, jnp.int32)
    seed = jnp.full((16,), 0x5F3759DF, dtype=jnp.int32) - lax.shift_right_logical(
        i, jnp.full((16,), 1, dtype=jnp.int32))
    x = lax.bitcast_convert_type(seed, jnp.float32)
    for _ in range(4):
        x = x * (1.5 - 0.5 * d * x * x)
    return x


def _deg_body(src_hbm, dst_hbm, ns_hbm, nd_hbm,
              idx_v, ones_v, zb_v, nb_v, acc_o, acc_i):
    c = lax.axis_index("c")
    s = lax.axis_index("s")

    zeros16 = jnp.zeros((16,), jnp.float32)
    ones16 = jnp.ones((16,), jnp.float32)
    for k in range(CH // 16):
        ones_v[pl.ds(k * 16, 16)] = ones16
    for k in range(RPSD // 16):
        zb_v[pl.ds(k * 16, 16)] = zeros16

    pltpu.sync_copy(zb_v, acc_o.at[pl.ds(s * RPSD, RPSD)])
    pltpu.sync_copy(zb_v, acc_i.at[pl.ds(s * RPSD, RPSD)])
    plsc.subcore_barrier()

    def _chunk(i, carry):
        base = s * EPC + i * CH
        pltpu.sync_copy(src_hbm.at[pl.ds(base, CH)], idx_v)
        pltpu.sync_copy(ones_v, acc_o.at[idx_v], add=True)
        pltpu.sync_copy(dst_hbm.at[pl.ds(base, CH)], idx_v)
        pltpu.sync_copy(ones_v, acc_i.at[idx_v], add=True)
        return carry
    lax.fori_loop(0, NCHD, _chunk, 0)

    plsc.subcore_barrier()

    base = c * (NP // NC) + s * NRM
    for sel in range(2):
        acc = acc_o if sel == 0 else acc_i
        out = ns_hbm if sel == 0 else nd_hbm
        pltpu.sync_copy(acc.at[pl.ds(base, NRM)], nb_v)
        for j in range(NRM // 16):
            d = jnp.maximum(nb_v[pl.ds(j * 16, 16)], 1.0)
            nb_v[pl.ds(j * 16, 16)] = _rsqrt16(d)
        pltpu.sync_copy(nb_v, out.at[pl.ds(base, NRM)])


_deg_kernel = pl.kernel(
    _deg_body,
    out_type=(jax.ShapeDtypeStruct((NP,), jnp.float32),
              jax.ShapeDtypeStruct((NP,), jnp.float32)),
    mesh=_MESH,
    scratch_types=[
        pltpu.VMEM((CH,), jnp.int32),
        pltpu.VMEM((CH,), jnp.float32),
        pltpu.VMEM((RPSD,), jnp.float32),
        pltpu.VMEM((NRM,), jnp.float32),
        pltpu.VMEM_SHARED((NP,), jnp.float32),
        pltpu.VMEM_SHARED((NP,), jnp.float32),
    ],
)


def _edge_body(hs_hbm, src_hbm, dst_hbm, out_hbm,
               sidx_v, didx_v, rows_v, zb_v, acc, sem):
    c = lax.axis_index("c")
    s = lax.axis_index("s")
    wid = c * NS + s

    zeros16 = jnp.zeros((16,), jnp.float32)

    def _zero_row(r, carry):
        for k in range(D // 16):
            zb_v[r, pl.ds(k * 16, 16)] = zeros16
        return carry
    lax.fori_loop(0, IC, _zero_row, 0)

    def _init(j, carry):
        idx = j * NS + s

        @pl.when(idx < NIC)
        def _():
            pltpu.sync_copy(zb_v, acc.at[pl.ds(idx * IC, IC)])
        return carry
    lax.fori_loop(0, ICL, _init, 0)
    plsc.subcore_barrier()

    def _chunk(i, carry):
        base = wid * EPW + i * CH
        pltpu.sync_copy(src_hbm.at[pl.ds(base, CH)], sidx_v)
        pltpu.sync_copy(dst_hbm.at[pl.ds(base, CH)], didx_v)
        pltpu.async_copy(hs_hbm.at[sidx_v], rows_v, sem).wait()
        pltpu.sync_copy(rows_v, acc.at[didx_v], add=True)
        return carry
    lax.fori_loop(0, NCH, _chunk, 0)

    plsc.subcore_barrier()

    def _writeout(j, carry):
        idx = j * NS + s

        @pl.when(idx < NIC)
        def _():
            pltpu.sync_copy(acc.at[pl.ds(idx * IC, IC)],
                            out_hbm.at[c].at[pl.ds(idx * IC, IC)])
        return carry
    lax.fori_loop(0, ICL, _writeout, 0)


_edge_kernel = pl.kernel(
    _edge_body,
    out_type=jax.ShapeDtypeStruct((NC, N, D), jnp.float32),
    mesh=_MESH,
    scratch_types=[
        pltpu.VMEM((CH,), jnp.int32),
        pltpu.VMEM((CH,), jnp.int32),
        pltpu.VMEM((CH, D), jnp.float32),
        pltpu.VMEM((IC, D), jnp.float32),
        pltpu.VMEM_SHARED((N, D), jnp.float32),
        pltpu.SemaphoreType.DMA,
    ],
)


BR = 1000


def _scale_mm_body(f_ref, w_ref, ns_ref, o_ref):
    h = jnp.dot(f_ref[...], w_ref[...], preferred_element_type=jnp.float32)
    o_ref[...] = h * ns_ref[...]


_scale_mm = pl.pallas_call(
    _scale_mm_body,
    grid=(N // BR,),
    in_specs=[
        pl.BlockSpec((BR, D), lambda i: (i, 0)),
        pl.BlockSpec((D, D), lambda i: (0, 0)),
        pl.BlockSpec((BR, 1), lambda i: (i, 0)),
    ],
    out_specs=pl.BlockSpec((BR, D), lambda i: (i, 0)),
    out_shape=jax.ShapeDtypeStruct((N, D), jnp.float32),
)


def _mid_body(aggp_ref, nd_ref, ns_ref, b_ref, w_ref, o_ref):
    agg = aggp_ref[0] + aggp_ref[1]
    x1 = jnp.maximum(agg * nd_ref[...] + b_ref[...], 0.0)
    h2 = jnp.dot(x1, w_ref[...], preferred_element_type=jnp.float32)
    o_ref[...] = h2 * ns_ref[...]


_mid = pl.pallas_call(
    _mid_body,
    grid=(N // BR,),
    in_specs=[
        pl.BlockSpec((NC, BR, D), lambda i: (0, i, 0)),
        pl.BlockSpec((BR, 1), lambda i: (i, 0)),
        pl.BlockSpec((BR, 1), lambda i: (i, 0)),
        pl.BlockSpec((1, D), lambda i: (0, 0)),
        pl.BlockSpec((D, D), lambda i: (0, 0)),
    ],
    out_specs=pl.BlockSpec((BR, D), lambda i: (i, 0)),
    out_shape=jax.ShapeDtypeStruct((N, D), jnp.float32),
)


def _final_body(aggp_ref, nd_ref, b_ref, o_ref):
    agg = aggp_ref[0] + aggp_ref[1]
    o_ref[...] = agg * nd_ref[...] + b_ref[...]


_final = pl.pallas_call(
    _final_body,
    grid=(N // BR,),
    in_specs=[
        pl.BlockSpec((NC, BR, D), lambda i: (0, i, 0)),
        pl.BlockSpec((BR, 1), lambda i: (i, 0)),
        pl.BlockSpec((1, D), lambda i: (0, 0)),
    ],
    out_specs=pl.BlockSpec((BR, D), lambda i: (i, 0)),
    out_shape=jax.ShapeDtypeStruct((N, D), jnp.float32),
)


def kernel(features, edge_index, W1, b1, W2, b2):
    ei = edge_index.astype(jnp.int32)
    src = ei[0]
    dst = ei[1]
    b1r = b1.reshape(1, D)
    b2r = b2.reshape(1, D)

    ns_f, nd_f = _deg_kernel(src, dst)
    ns = ns_f[:N].reshape(N, 1)
    nd = nd_f[:N].reshape(N, 1)
    hs1 = _scale_mm(features, W1, ns)
    agg1p = _edge_kernel(hs1, src, dst)
    hs2 = _mid(agg1p, nd, ns, b1r, W2)
    agg2p = _edge_kernel(hs2, src, dst)
    return _final(agg2p, nd, b2r)


# speedup vs baseline: 8.2992x; 1.3689x over previous
import functools

import jax
import jax.numpy as jnp
from jax import lax
from jax.experimental import pallas as pl
from jax.experimental.pallas import tpu as pltpu
from jax.experimental.pallas import tpu_sc as plsc

N = 10000
E = 320000
D = 128
NC = 2
NS = 16
NW = NC * NS
EPW = E // NW
CH = 80
NCH = EPW // CH
IC = 80
NIC = N // IC
ICL = -(-NIC // NS)

_MESH = plsc.VectorSubcoreMesh(
    core_axis_name="c", subcore_axis_name="s", num_cores=NC, num_subcores=NS)


NP = 10240
RPSD = NP // NS
EPC = E // NS
NCHD = EPC // CH
NRM = NP // NC // NS


def _rsqrt16(d):
    i = lax.bitcast_convert_type(d, jnp.int32)
    seed = jnp.full((16,), 0x5F3759DF, dtype=jnp.int32) - lax.shift_right_logical(
        i, jnp.full((16,), 1, dtype=jnp.int32))
    x = lax.bitcast_convert_type(seed, jnp.float32)
    for _ in range(4):
        x = x * (1.5 - 0.5 * d * x * x)
    return x


def _deg_body(src_hbm, dst_hbm, ns_hbm, nd_hbm,
              idx_v, ones_v, zb_v, nb_v, acc_o, acc_i):
    c = lax.axis_index("c")
    s = lax.axis_index("s")

    zeros16 = jnp.zeros((16,), jnp.float32)
    ones16 = jnp.ones((16,), jnp.float32)
    for k in range(CH // 16):
        ones_v[pl.ds(k * 16, 16)] = ones16
    for k in range(RPSD // 16):
        zb_v[pl.ds(k * 16, 16)] = zeros16

    pltpu.sync_copy(zb_v, acc_o.at[pl.ds(s * RPSD, RPSD)])
    pltpu.sync_copy(zb_v, acc_i.at[pl.ds(s * RPSD, RPSD)])
    plsc.subcore_barrier()

    def _chunk(i, carry):
        base = s * EPC + i * CH
        pltpu.sync_copy(src_hbm.at[pl.ds(base, CH)], idx_v)
        pltpu.sync_copy(ones_v, acc_o.at[idx_v], add=True)
        pltpu.sync_copy(dst_hbm.at[pl.ds(base, CH)], idx_v)
        pltpu.sync_copy(ones_v, acc_i.at[idx_v], add=True)
        return carry
    lax.fori_loop(0, NCHD, _chunk, 0)

    plsc.subcore_barrier()

    base = c * (NP // NC) + s * NRM
    for sel in range(2):
        acc = acc_o if sel == 0 else acc_i
        out = ns_hbm if sel == 0 else nd_hbm
        pltpu.sync_copy(acc.at[pl.ds(base, NRM)], nb_v)
        for j in range(NRM // 16):
            d = jnp.maximum(nb_v[pl.ds(j * 16, 16)], 1.0)
            nb_v[pl.ds(j * 16, 16)] = _rsqrt16(d)
        pltpu.sync_copy(nb_v, out.at[pl.ds(base, NRM)])


_deg_kernel = pl.kernel(
    _deg_body,
    out_type=(jax.ShapeDtypeStruct((NP,), jnp.float32),
              jax.ShapeDtypeStruct((NP,), jnp.float32)),
    mesh=_MESH,
    scratch_types=[
        pltpu.VMEM((CH,), jnp.int32),
        pltpu.VMEM((CH,), jnp.float32),
        pltpu.VMEM((RPSD,), jnp.float32),
        pltpu.VMEM((NRM,), jnp.float32),
        pltpu.VMEM_SHARED((NP,), jnp.float32),
        pltpu.VMEM_SHARED((NP,), jnp.float32),
    ],
)


def _edge_body(hs_hbm, src_hbm, dst_hbm, out_hbm,
               sidx0, didx0, sidx1, didx1, rows0, rows1, zb_v, acc,
               sem0, sem1):
    c = lax.axis_index("c")
    s = lax.axis_index("s")
    wid = c * NS + s

    zeros16 = jnp.zeros((16,), jnp.float32)

    def _zero_row(r, carry):
        for k in range(D // 16):
            zb_v[r, pl.ds(k * 16, 16)] = zeros16
        return carry
    lax.fori_loop(0, IC, _zero_row, 0)

    def _init(j, carry):
        idx = j * NS + s

        @pl.when(idx < NIC)
        def _():
            pltpu.sync_copy(zb_v, acc.at[pl.ds(idx * IC, IC)])
        return carry
    lax.fori_loop(0, ICL, _init, 0)
    plsc.subcore_barrier()

    ebase = wid * EPW

    def _load_idx(ci, sidx, didx):
        pltpu.sync_copy(src_hbm.at[pl.ds(ebase + ci * CH, CH)], sidx)
        pltpu.sync_copy(dst_hbm.at[pl.ds(ebase + ci * CH, CH)], didx)

    _load_idx(0, sidx0, didx0)
    pltpu.async_copy(hs_hbm.at[sidx0], rows0, sem0)

    def _pair(k, carry):
        _load_idx(2 * k + 1, sidx1, didx1)
        pltpu.async_copy(hs_hbm.at[sidx1], rows1, sem1)
        pltpu.make_async_copy(hs_hbm.at[sidx0], rows0, sem0).wait()
        pltpu.sync_copy(rows0, acc.at[didx0], add=True)
        _load_idx(2 * k + 2, sidx0, didx0)
        pltpu.async_copy(hs_hbm.at[sidx0], rows0, sem0)
        pltpu.make_async_copy(hs_hbm.at[sidx1], rows1, sem1).wait()
        pltpu.sync_copy(rows1, acc.at[didx1], add=True)
        return carry
    lax.fori_loop(0, (NCH - 1) // 2, _pair, 0)
    pltpu.make_async_copy(hs_hbm.at[sidx0], rows0, sem0).wait()
    pltpu.sync_copy(rows0, acc.at[didx0], add=True)

    plsc.subcore_barrier()

    def _writeout(j, carry):
        idx = j * NS + s

        @pl.when(idx < NIC)
        def _():
            pltpu.sync_copy(acc.at[pl.ds(idx * IC, IC)],
                            out_hbm.at[c].at[pl.ds(idx * IC, IC)])
        return carry
    lax.fori_loop(0, ICL, _writeout, 0)


_edge_kernel = pl.kernel(
    _edge_body,
    out_type=jax.ShapeDtypeStruct((NC, N, D), jnp.float32),
    mesh=_MESH,
    scratch_types=[
        pltpu.VMEM((CH,), jnp.int32),
        pltpu.VMEM((CH,), jnp.int32),
        pltpu.VMEM((CH,), jnp.int32),
        pltpu.VMEM((CH,), jnp.int32),
        pltpu.VMEM((CH, D), jnp.float32),
        pltpu.VMEM((CH, D), jnp.float32),
        pltpu.VMEM((IC, D), jnp.float32),
        pltpu.VMEM_SHARED((N, D), jnp.float32),
        pltpu.SemaphoreType.DMA,
        pltpu.SemaphoreType.DMA,
    ],
)


BR = 1000


def _scale_mm_body(f_ref, w_ref, ns_ref, o_ref):
    h = jnp.dot(f_ref[...], w_ref[...], preferred_element_type=jnp.float32)
    o_ref[...] = h * ns_ref[...]


_scale_mm = pl.pallas_call(
    _scale_mm_body,
    grid=(N // BR,),
    in_specs=[
        pl.BlockSpec((BR, D), lambda i: (i, 0)),
        pl.BlockSpec((D, D), lambda i: (0, 0)),
        pl.BlockSpec((BR, 1), lambda i: (i, 0)),
    ],
    out_specs=pl.BlockSpec((BR, D), lambda i: (i, 0)),
    out_shape=jax.ShapeDtypeStruct((N, D), jnp.float32),
)


def _mid_body(aggp_ref, nd_ref, ns_ref, b_ref, w_ref, o_ref):
    agg = aggp_ref[0] + aggp_ref[1]
    x1 = jnp.maximum(agg * nd_ref[...] + b_ref[...], 0.0)
    h2 = jnp.dot(x1, w_ref[...], preferred_element_type=jnp.float32)
    o_ref[...] = h2 * ns_ref[...]


_mid = pl.pallas_call(
    _mid_body,
    grid=(N // BR,),
    in_specs=[
        pl.BlockSpec((NC, BR, D), lambda i: (0, i, 0)),
        pl.BlockSpec((BR, 1), lambda i: (i, 0)),
        pl.BlockSpec((BR, 1), lambda i: (i, 0)),
        pl.BlockSpec((1, D), lambda i: (0, 0)),
        pl.BlockSpec((D, D), lambda i: (0, 0)),
    ],
    out_specs=pl.BlockSpec((BR, D), lambda i: (i, 0)),
    out_shape=jax.ShapeDtypeStruct((N, D), jnp.float32),
)


def _final_body(aggp_ref, nd_ref, b_ref, o_ref):
    agg = aggp_ref[0] + aggp_ref[1]
    o_ref[...] = agg * nd_ref[...] + b_ref[...]


_final = pl.pallas_call(
    _final_body,
    grid=(N // BR,),
    in_specs=[
        pl.BlockSpec((NC, BR, D), lambda i: (0, i, 0)),
        pl.BlockSpec((BR, 1), lambda i: (i, 0)),
        pl.BlockSpec((1, D), lambda i: (0, 0)),
    ],
    out_specs=pl.BlockSpec((BR, D), lambda i: (i, 0)),
    out_shape=jax.ShapeDtypeStruct((N, D), jnp.float32),
)


def kernel(features, edge_index, W1, b1, W2, b2):
    ei = edge_index.astype(jnp.int32)
    src = ei[0]
    dst = ei[1]
    b1r = b1.reshape(1, D)
    b2r = b2.reshape(1, D)

    ns_f, nd_f = _deg_kernel(src, dst)
    ns = ns_f[:N].reshape(N, 1)
    nd = nd_f[:N].reshape(N, 1)
    hs1 = _scale_mm(features, W1, ns)
    agg1p = _edge_kernel(hs1, src, dst)
    hs2 = _mid(agg1p, nd, ns, b1r, W2)
    agg2p = _edge_kernel(hs2, src, dst)
    return _final(agg2p, nd, b2r)


# speedup vs baseline: 20.4116x; 2.4595x over previous
import jax
import jax.numpy as jnp
from jax import lax
from jax.experimental import pallas as pl
from jax.experimental.pallas import tpu as pltpu
from jax.experimental.pallas import tpu_sc as plsc

N = 10000
E = 320000
D = 128
NC = 2
NS = 16
NW = NC * NS
EPW = E // NW
CH = 80
NCH = EPW // CH
IC = 80
NIC = N // IC
ICL = -(-NIC // NS)

_MESH = plsc.VectorSubcoreMesh(
    core_axis_name="c", subcore_axis_name="s", num_cores=NC, num_subcores=NS)


NP = 10240
RPSD = NP // NS
EPC = E // NS
NCHD = EPC // CH
NRM = NP // NC // NS


def _rsqrt16(d):
    i = lax.bitcast_convert_type(d, jnp.int32)
    seed = jnp.full((16,), 0x5F3759DF, dtype=jnp.int32) - lax.shift_right_logical(
        i, jnp.full((16,), 1, dtype=jnp.int32))
    x = lax.bitcast_convert_type(seed, jnp.float32)
    for _ in range(4):
        x = x * (1.5 - 0.5 * d * x * x)
    return x


DEGQ = 4


def _deg_body(src_hbm, dst_hbm, ns_hbm, nd_hbm,
              sall, dall, ones_v, zb_v, nb_v,
              acc_o, acc_i, semA, semB, semS, semD):
    c = lax.axis_index("c")
    s = lax.axis_index("s")

    pltpu.async_copy(src_hbm.at[pl.ds(s * EPC, EPC)], sall, semA)
    pltpu.async_copy(dst_hbm.at[pl.ds(s * EPC, EPC)], dall, semB)

    zeros16 = jnp.zeros((16,), jnp.float32)
    ones16 = jnp.ones((16,), jnp.float32)
    for k in range(CH // 16):
        ones_v[pl.ds(k * 16, 16)] = ones16
    for k in range(RPSD // 16):
        zb_v[pl.ds(k * 16, 16)] = zeros16

    pltpu.sync_copy(zb_v, acc_o.at[pl.ds(s * RPSD, RPSD)])
    pltpu.sync_copy(zb_v, acc_i.at[pl.ds(s * RPSD, RPSD)])
    pltpu.make_async_copy(src_hbm.at[pl.ds(s * EPC, EPC)], sall, semA).wait()
    pltpu.make_async_copy(dst_hbm.at[pl.ds(s * EPC, EPC)], dall, semB).wait()
    plsc.subcore_barrier()

    def _fire(j):
        pltpu.async_copy(ones_v, acc_o.at[sall.at[pl.ds(j * CH, CH)]], semS,
                         add=True)
        pltpu.async_copy(ones_v, acc_i.at[dall.at[pl.ds(j * CH, CH)]], semD,
                         add=True)

    def _drain(j):
        pltpu.make_async_copy(ones_v, acc_o.at[sall.at[pl.ds(j * CH, CH)]],
                              semS).wait()
        pltpu.make_async_copy(ones_v, acc_i.at[dall.at[pl.ds(j * CH, CH)]],
                              semD).wait()

    for j in range(DEGQ):
        _fire(j)

    def _step(j, carry):
        _drain(j)
        _fire(j + DEGQ)
        return carry
    lax.fori_loop(0, NCHD - DEGQ, _step, 0)
    for j in range(DEGQ):
        _drain(NCHD - DEGQ + j)

    plsc.subcore_barrier()

    base = c * (NP // NC) + s * NRM
    for sel in range(2):
        acc = acc_o if sel == 0 else acc_i
        out = ns_hbm if sel == 0 else nd_hbm
        pltpu.sync_copy(acc.at[pl.ds(base, NRM)], nb_v)
        for j in range(NRM // 16):
            d = jnp.maximum(nb_v[pl.ds(j * 16, 16)], 1.0)
            nb_v[pl.ds(j * 16, 16)] = _rsqrt16(d)
        pltpu.sync_copy(nb_v, out.at[pl.ds(base, NRM)])


_deg_kernel = pl.kernel(
    _deg_body,
    out_type=(jax.ShapeDtypeStruct((NP,), jnp.float32),
              jax.ShapeDtypeStruct((NP,), jnp.float32)),
    mesh=_MESH,
    scratch_types=[
        pltpu.VMEM((EPC,), jnp.int32),
        pltpu.VMEM((EPC,), jnp.int32),
        pltpu.VMEM((CH,), jnp.float32),
        pltpu.VMEM((RPSD,), jnp.float32),
        pltpu.VMEM((NRM,), jnp.float32),
        pltpu.VMEM_SHARED((NP,), jnp.float32),
        pltpu.VMEM_SHARED((NP,), jnp.float32),
        pltpu.SemaphoreType.DMA,
        pltpu.SemaphoreType.DMA,
        pltpu.SemaphoreType.DMA,
        pltpu.SemaphoreType.DMA,
    ],
)


def _edge_body(hs_hbm, src_hbm, dst_hbm, out_hbm,
               sall, dall, rows0, rows1, rows2, acc,
               g0, g1, g2, w0, w1, w2, semA, semB):
    c = lax.axis_index("c")
    s = lax.axis_index("s")
    wid = c * NS + s
    ebase = wid * EPW
    rows = (rows0, rows1, rows2)
    gs = (g0, g1, g2)
    ws = (w0, w1, w2)

    pltpu.async_copy(src_hbm.at[pl.ds(ebase, EPW)], sall, semA)
    pltpu.async_copy(dst_hbm.at[pl.ds(ebase, EPW)], dall, semB)

    zeros16 = jnp.zeros((16,), jnp.float32)

    def _zero_row(r, carry):
        for k in range(D // 16):
            rows0[r, pl.ds(k * 16, 16)] = zeros16
        return carry
    lax.fori_loop(0, IC, _zero_row, 0)

    def _init(j, carry):
        idx = j * NS + s

        @pl.when(idx < NIC)
        def _():
            pltpu.sync_copy(rows0, acc.at[pl.ds(idx * IC, IC)])
        return carry
    lax.fori_loop(0, ICL, _init, 0)
    pltpu.make_async_copy(src_hbm.at[pl.ds(ebase, EPW)], sall, semA).wait()
    pltpu.make_async_copy(dst_hbm.at[pl.ds(ebase, EPW)], dall, semB).wait()
    plsc.subcore_barrier()

    def _sidx(j):
        return sall.at[pl.ds(j * CH, CH)]

    def _didx(j):
        return dall.at[pl.ds(j * CH, CH)]

    def _fire_g(j, b):
        pltpu.async_copy(hs_hbm.at[_sidx(j)], rows[b], gs[b])

    def _wait_g(j, b):
        pltpu.make_async_copy(hs_hbm.at[_sidx(j)], rows[b], gs[b]).wait()

    def _fire_w(j, b):
        pltpu.async_copy(rows[b], acc.at[_didx(j)], ws[b], add=True)

    def _wait_w(j, b):
        pltpu.make_async_copy(rows[b], acc.at[_didx(j)], ws[b]).wait()

    def _visit(j, b, first, last):
        if not first:

            @pl.when(j >= 2)
            def _():
                _wait_w(j - 2, (b + 1) % 3)
        if not last:

            @pl.when(j + 1 < NCH)
            def _():
                _fire_g(j + 1, (b + 1) % 3)
        _wait_g(j, b)
        _fire_w(j, b)

    _fire_g(0, 0)

    def _group(t, carry):
        for b in range(3):
            _visit(3 * t + b, b, first=False, last=False)
        return carry
    NT = (NCH - 2) // 3
    lax.fori_loop(0, NT, _group, 0)
    _visit(NCH - 2, (NCH - 2) % 3, first=False, last=False)
    _visit(NCH - 1, (NCH - 1) % 3, first=False, last=True)
    _wait_w(NCH - 2, (NCH - 2) % 3)
    _wait_w(NCH - 1, (NCH - 1) % 3)

    plsc.subcore_barrier()

    def _writeout(j, carry):
        idx = j * NS + s

        @pl.when(idx < NIC)
        def _():
            pltpu.sync_copy(acc.at[pl.ds(idx * IC, IC)],
                            out_hbm.at[c].at[pl.ds(idx * IC, IC)])
        return carry
    lax.fori_loop(0, ICL, _writeout, 0)


_edge_kernel = pl.kernel(
    _edge_body,
    out_type=jax.ShapeDtypeStruct((NC, N, D), jnp.float32),
    mesh=_MESH,
    scratch_types=[
        pltpu.VMEM((EPW,), jnp.int32),
        pltpu.VMEM((EPW,), jnp.int32),
        pltpu.VMEM((CH, D), jnp.float32),
        pltpu.VMEM((CH, D), jnp.float32),
        pltpu.VMEM((CH, D), jnp.float32),
        pltpu.VMEM_SHARED((N, D), jnp.float32),
        pltpu.SemaphoreType.DMA,
        pltpu.SemaphoreType.DMA,
        pltpu.SemaphoreType.DMA,
        pltpu.SemaphoreType.DMA,
        pltpu.SemaphoreType.DMA,
        pltpu.SemaphoreType.DMA,
        pltpu.SemaphoreType.DMA,
        pltpu.SemaphoreType.DMA,
    ],
)


BR = 1000


def _scale_mm_body(f_ref, w_ref, ns_ref, o_ref):
    h = jnp.dot(f_ref[...], w_ref[...], preferred_element_type=jnp.float32)
    o_ref[...] = h * ns_ref[...]


_scale_mm = pl.pallas_call(
    _scale_mm_body,
    grid=(N // BR,),
    in_specs=[
        pl.BlockSpec((BR, D), lambda i: (i, 0)),
        pl.BlockSpec((D, D), lambda i: (0, 0)),
        pl.BlockSpec((BR, 1), lambda i: (i, 0)),
    ],
    out_specs=pl.BlockSpec((BR, D), lambda i: (i, 0)),
    out_shape=jax.ShapeDtypeStruct((N, D), jnp.float32),
)


def _mid_body(aggp_ref, nd_ref, ns_ref, b_ref, w_ref, o_ref):
    agg = aggp_ref[0] + aggp_ref[1]
    x1 = jnp.maximum(agg * nd_ref[...] + b_ref[...], 0.0)
    h2 = jnp.dot(x1, w_ref[...], preferred_element_type=jnp.float32)
    o_ref[...] = h2 * ns_ref[...]


_mid = pl.pallas_call(
    _mid_body,
    grid=(N // BR,),
    in_specs=[
        pl.BlockSpec((NC, BR, D), lambda i: (0, i, 0)),
        pl.BlockSpec((BR, 1), lambda i: (i, 0)),
        pl.BlockSpec((BR, 1), lambda i: (i, 0)),
        pl.BlockSpec((1, D), lambda i: (0, 0)),
        pl.BlockSpec((D, D), lambda i: (0, 0)),
    ],
    out_specs=pl.BlockSpec((BR, D), lambda i: (i, 0)),
    out_shape=jax.ShapeDtypeStruct((N, D), jnp.float32),
)


def _final_body(aggp_ref, nd_ref, b_ref, o_ref):
    agg = aggp_ref[0] + aggp_ref[1]
    o_ref[...] = agg * nd_ref[...] + b_ref[...]


_final = pl.pallas_call(
    _final_body,
    grid=(N // BR,),
    in_specs=[
        pl.BlockSpec((NC, BR, D), lambda i: (0, i, 0)),
        pl.BlockSpec((BR, 1), lambda i: (i, 0)),
        pl.BlockSpec((1, D), lambda i: (0, 0)),
    ],
    out_specs=pl.BlockSpec((BR, D), lambda i: (i, 0)),
    out_shape=jax.ShapeDtypeStruct((N, D), jnp.float32),
)


def kernel(features, edge_index, W1, b1, W2, b2):
    ei = edge_index.astype(jnp.int32)
    src = ei[0]
    dst = ei[1]
    b1r = b1.reshape(1, D)
    b2r = b2.reshape(1, D)

    ns_f, nd_f = _deg_kernel(src, dst)
    ns = ns_f[:N].reshape(N, 1)
    nd = nd_f[:N].reshape(N, 1)
    hs1 = _scale_mm(features, W1, ns)
    agg1p = _edge_kernel(hs1, src, dst)
    hs2 = _mid(agg1p, nd, ns, b1r, W2)
    agg2p = _edge_kernel(hs2, src, dst)
    return _final(agg2p, nd, b2r)
